# baseline (speedup 1.0000x reference)
import jax
import jax.numpy as jnp
from jax.experimental import pallas as pl

B = 1024
P = 10
LOUT = 21
D = 1024

BB = 128
RB = 8


def _fill_body(vtok_ref, end_ref, rep_ref, canvas_ref, out_ref):
    del canvas_ref
    j = pl.program_id(1)

    @pl.when(j == 0)
    def _rows_0_7():
        out_ref[...] = vtok_ref[...]

    @pl.when(j == 1)
    def _rows_8_15():
        out_ref[...] = jnp.concatenate(
            [vtok_ref[:, 0:2, :],
             jnp.broadcast_to(end_ref[...][None, :, :], (BB, 1, D)),
             jnp.broadcast_to(rep_ref[...][None, :, :], (BB, 5, D))],
            axis=1)


def kernel(categories, vtok, end, zero):
    rep = jnp.where(categories[0, 11] == 0, zero, end)
    sel = jnp.where(jnp.arange(LOUT)[:, None] == P, end, rep)
    canvas = jnp.broadcast_to(sel[None], (B, LOUT, D))
    return pl.pallas_call(
        _fill_body,
        grid=(B // BB, 2),
        in_specs=[
            pl.BlockSpec((BB, RB, D), lambda i, j: (i, j, 0)),
            pl.BlockSpec((1, D), lambda i, j: (0, 0)),
            pl.BlockSpec((1, D), lambda i, j: (0, 0)),
            pl.BlockSpec(memory_space=pl.MemorySpace.ANY),
        ],
        out_specs=pl.BlockSpec((BB, RB, D), lambda i, j: (i, j, 0)),
        out_shape=jax.ShapeDtypeStruct((B, LOUT, D), jnp.float32),
        input_output_aliases={3: 0},
    )(vtok, end, rep, canvas)
